# baseline (speedup 1.0000x reference)
import jax
import jax.numpy as jnp
from jax.experimental import pallas as pl
from jax.experimental.pallas import tpu as pltpu


def _copy_body(in_ref, out_ref, sem):
    c = pltpu.make_async_copy(in_ref, out_ref, sem)
    c.start()
    c.wait()


def kernel(item_emb):
    return pl.pallas_call(
        _copy_body,
        in_specs=[pl.BlockSpec(memory_space=pl.ANY)],
        out_specs=pl.BlockSpec(memory_space=pl.ANY),
        out_shape=jax.ShapeDtypeStruct(item_emb.shape, item_emb.dtype),
        scratch_shapes=[pltpu.SemaphoreType.DMA],
    )(item_emb)


# speedup vs baseline: 17.9645x; 17.9645x over previous
import jax
import jax.numpy as jnp
from jax.experimental import pallas as pl
from jax.experimental.pallas import tpu as pltpu

_BLOCK_ROWS = 8000


def _copy_block(in_ref, out_ref):
    out_ref[...] = in_ref[...]


def kernel(item_emb):
    n, d = item_emb.shape
    return pl.pallas_call(
        _copy_block,
        grid=(n // _BLOCK_ROWS,),
        in_specs=[pl.BlockSpec((_BLOCK_ROWS, d), lambda i: (i, 0))],
        out_specs=pl.BlockSpec((_BLOCK_ROWS, d), lambda i: (i, 0)),
        out_shape=jax.ShapeDtypeStruct(item_emb.shape, item_emb.dtype),
    )(item_emb)
